# baseline (speedup 1.0000x reference)
import functools

import jax
import jax.numpy as jnp
from jax import lax
from jax.experimental import pallas as pl
from jax.experimental.pallas import tpu as pltpu
from jax.experimental.pallas import tpu_sc as plsc

_F32 = jnp.float32
_NC = 2
_NS = 16
_NW = _NC * _NS
_CH = 96


def _prep_body(x_ref, wmsg_ref, wa_ref, wb_ref, w1_ref, ba_ref, bb_ref,
               b1_ref, xm_ref, wfa_ref, wfb_ref, bf_ref):
    xm_ref[...] = jnp.dot(x_ref[...], wmsg_ref[...],
                          preferred_element_type=_F32)
    w1a = w1_ref[256:512, :]
    w1b = w1_ref[512:768, :]
    wfa_ref[...] = jnp.dot(wa_ref[...], w1a, preferred_element_type=_F32)
    wfb_ref[...] = jnp.dot(wb_ref[...], w1b, preferred_element_type=_F32)
    bf_ref[...] = (b1_ref[...]
                   + jnp.dot(ba_ref[...], w1a, preferred_element_type=_F32)
                   + jnp.dot(bb_ref[...], w1b, preferred_element_type=_F32))


def _prep(x, W_msg, Wa, Wb, W1, ba, bb, b1):
    n, d = x.shape
    k = Wa.shape[0]
    return pl.pallas_call(
        _prep_body,
        out_shape=(
            jax.ShapeDtypeStruct((n, d), _F32),
            jax.ShapeDtypeStruct((k, 128), _F32),
            jax.ShapeDtypeStruct((k, 128), _F32),
            jax.ShapeDtypeStruct((1, 128), _F32),
        ),
    )(x, W_msg, Wa, Wb, W1, ba.reshape(1, -1), bb.reshape(1, -1),
      b1.reshape(1, -1))


def _mid_body(agg_ref, wupd_ref, bupd_ref, w1_ref, pa_ref, pb_ref):
    s = agg_ref[0] + agg_ref[1]
    r = jnp.maximum(
        jnp.dot(s, wupd_ref[...], preferred_element_type=_F32)
        + bupd_ref[...], 0.0)
    pa_ref[...] = jnp.dot(r, w1_ref[0:128, :], preferred_element_type=_F32)
    pb_ref[...] = jnp.dot(r, w1_ref[128:256, :], preferred_element_type=_F32)


def _mid(agg2, W_upd, b_upd, W1):
    n = agg2.shape[1]
    return pl.pallas_call(
        _mid_body,
        out_shape=(
            jax.ShapeDtypeStruct((n, 128), _F32),
            jax.ShapeDtypeStruct((n, 128), _F32),
        ),
    )(agg2, W_upd, b_upd.reshape(1, -1), W1)


def _esm_body(ea_ref, eb_ref, wfa_ref, wfb_ref, bf_ref, out_ref):
    out_ref[...] = (
        jnp.dot(ea_ref[...], wfa_ref[...], preferred_element_type=_F32)
        + jnp.dot(eb_ref[...], wfb_ref[...], preferred_element_type=_F32)
        + bf_ref[...])


def _esm(ESMa, ESMb, WFa, WFb, bfold):
    b, k = ESMa.shape
    bm = 1024
    grid = (b // bm,)
    return pl.pallas_call(
        _esm_body,
        grid=grid,
        in_specs=[
            pl.BlockSpec((bm, k), lambda i: (i, 0)),
            pl.BlockSpec((bm, k), lambda i: (i, 0)),
            pl.BlockSpec((k, 128), lambda i: (0, 0)),
            pl.BlockSpec((k, 128), lambda i: (0, 0)),
            pl.BlockSpec((1, 128), lambda i: (0, 0)),
        ],
        out_specs=pl.BlockSpec((bm, 128), lambda i: (i, 0)),
        out_shape=jax.ShapeDtypeStruct((b, 128), _F32),
    )(ESMa, ESMb, WFa, WFb, bfold)


def _head_body(ga_ref, gb_ref, ep_ref, w2_ref, b2_ref, out_ref):
    h = jnp.maximum(ga_ref[...] + gb_ref[...] + ep_ref[...], 0.0)
    out_ref[...] = (jnp.dot(h, w2_ref[...], preferred_element_type=_F32)
                    + b2_ref[...])


def _head(GA, GB, ESMpart, W2, b2):
    b = GA.shape[0]
    ncls = W2.shape[1]
    bm = 2048
    grid = (b // bm,)
    return pl.pallas_call(
        _head_body,
        grid=grid,
        in_specs=[
            pl.BlockSpec((bm, 128), lambda i: (i, 0)),
            pl.BlockSpec((bm, 128), lambda i: (i, 0)),
            pl.BlockSpec((bm, 128), lambda i: (i, 0)),
            pl.BlockSpec((128, ncls), lambda i: (0, 0)),
            pl.BlockSpec((1, ncls), lambda i: (0, 0)),
        ],
        out_specs=pl.BlockSpec((bm, ncls), lambda i: (i, 0)),
        out_shape=jax.ShapeDtypeStruct((b, ncls), _F32),
    )(GA, GB, ESMpart, W2, b2.reshape(1, -1))


_SG = 12
_NB = 3


def _edge_agg(xm, src4d, dst4d, zeros_nd):
    npad = zeros_nd.shape[0]
    nstripe = src4d.shape[1]
    rpt = npad // _NS
    mesh = plsc.VectorSubcoreMesh(core_axis_name="c", subcore_axis_name="s",
                                  num_cores=_NC, num_subcores=_NS)

    @functools.partial(
        pl.kernel,
        out_type=jax.ShapeDtypeStruct((_NC, npad, 128), _F32),
        mesh=mesh,
        scratch_types=[
            pltpu.VMEM((2, _SG, _CH), jnp.int32),
            pltpu.VMEM((2, _SG, _CH), jnp.int32),
        ] + [pltpu.VMEM((_CH, 128), _F32)] * _NB
          + [pltpu.SemaphoreType.DMA] * (2 * _NB)
          + [pltpu.SemaphoreType.DMA]
          + [pltpu.VMEM_SHARED((npad, 128), _F32)],
    )
    def k(xm_hbm, src_hbm, dst_hbm, zero_hbm, agg_hbm, srcv, dstv, *scr):
        rows = scr[:_NB]
        gsem = scr[_NB:2 * _NB]
        ssem = scr[2 * _NB:3 * _NB]
        isem = scr[3 * _NB]
        acc = scr[3 * _NB + 1]
        c = lax.axis_index("c")
        s = lax.axis_index("s")
        w = c * _NS + s
        pltpu.sync_copy(zero_hbm.at[pl.ds(s * rpt, rpt)],
                        acc.at[pl.ds(s * rpt, rpt)])
        plsc.subcore_barrier()
        pltpu.sync_copy(src_hbm.at[w].at[0], srcv.at[0])
        pltpu.sync_copy(dst_hbm.at[w].at[0], dstv.at[0])

        def stripe(t, carry):
            slot = lax.rem(t, 2)
            nxt = lax.rem(t + 1, 2)
            sv = srcv.at[slot]
            dv = dstv.at[slot]

            @pl.when(t + 1 < nstripe)
            def _():
                pltpu.async_copy(src_hbm.at[w].at[t + 1], srcv.at[nxt], isem)
                pltpu.async_copy(dst_hbm.at[w].at[t + 1], dstv.at[nxt], isem)

            pltpu.async_copy(xm_hbm.at[sv.at[0]], rows[0], gsem[0])
            pltpu.async_copy(xm_hbm.at[sv.at[1]], rows[1], gsem[1])

            def chunk3(g3, carry2):
                for b3 in range(_NB):
                    j = _NB * g3 + b3
                    b = b3
                    bn = (b3 + 2) % _NB
                    pltpu.make_async_copy(xm_hbm.at[sv.at[j]], rows[b],
                                          gsem[b]).wait()
                    pltpu.async_copy(rows[b], acc.at[dv.at[j]], ssem[b],
                                     add=True)

                    @pl.when(j + 2 < _SG)
                    def _():
                        @pl.when(j >= 1)
                        def _():
                            pltpu.make_async_copy(
                                rows[bn], acc.at[pl.ds(0, _CH)],
                                ssem[bn]).wait()
                        pltpu.async_copy(xm_hbm.at[sv.at[j + 2]], rows[bn],
                                         gsem[bn])
                return carry2

            lax.fori_loop(0, _SG // _NB, chunk3, 0)
            for j in (_SG - 3, _SG - 2, _SG - 1):
                pltpu.make_async_copy(rows[j % _NB], acc.at[pl.ds(0, _CH)],
                                      ssem[j % _NB]).wait()

            @pl.when(t + 1 < nstripe)
            def _():
                pltpu.make_async_copy(src_hbm.at[w].at[t + 1], srcv.at[nxt],
                                      isem).wait()
                pltpu.make_async_copy(dst_hbm.at[w].at[t + 1], dstv.at[nxt],
                                      isem).wait()

            return carry

        lax.fori_loop(0, nstripe, stripe, 0)
        plsc.subcore_barrier()
        pltpu.sync_copy(acc.at[pl.ds(s * rpt, rpt)],
                        agg_hbm.at[c].at[pl.ds(s * rpt, rpt)])

    return k(xm, src4d, dst4d, zeros_nd)


def _gather_pred(Pa, Pb, ia2d, ib2d):
    b = ia2d.shape[0] * ia2d.shape[1] * ia2d.shape[2]
    nchunk_w = ia2d.shape[1]
    gch = ia2d.shape[2]
    mesh = plsc.VectorSubcoreMesh(core_axis_name="c", subcore_axis_name="s",
                                  num_cores=_NC, num_subcores=_NS)

    @functools.partial(
        pl.kernel,
        out_type=(jax.ShapeDtypeStruct((b, 128), _F32),
                  jax.ShapeDtypeStruct((b, 128), _F32)),
        mesh=mesh,
        scratch_types=[
            pltpu.VMEM((nchunk_w, gch), jnp.int32),
            pltpu.VMEM((nchunk_w, gch), jnp.int32),
            pltpu.VMEM((gch, 128), _F32),
            pltpu.VMEM((gch, 128), _F32),
            pltpu.SemaphoreType.DMA,
            pltpu.SemaphoreType.DMA,
        ],
    )
    def k(pa_hbm, pb_hbm, ia_hbm, ib_hbm, ga_hbm, gb_hbm,
          iav, ibv, bufa, bufb, sema, semb):
        c = lax.axis_index("c")
        s = lax.axis_index("s")
        w = c * _NS + s
        base = w * nchunk_w
        pltpu.sync_copy(ia_hbm.at[w], iav)
        pltpu.sync_copy(ib_hbm.at[w], ibv)

        def body(j, carry):
            ca = pltpu.async_copy(pa_hbm.at[iav.at[j]], bufa, sema)
            cb = pltpu.async_copy(pb_hbm.at[ibv.at[j]], bufb, semb)
            ca.wait()
            cb.wait()
            row0 = (base + j) * gch
            pltpu.sync_copy(bufa, ga_hbm.at[pl.ds(row0, gch)])
            pltpu.sync_copy(bufb, gb_hbm.at[pl.ds(row0, gch)])
            return carry

        lax.fori_loop(0, nchunk_w, body, 0)

    return k(Pa, Pb, ia2d, ib2d)


def kernel(x, edge_index, edge_a, edge_b, ESM_nodea_emb, ESM_nodeb_emb,
           W_msg, W_upd, b_upd, Wa, ba, Wb, bb, W1, b1, W2, b2):
    n = x.shape[0]
    e = edge_index.shape[1]
    bsz = edge_a.shape[0]

    xm, WFa, WFb, bfold = _prep(x, W_msg, Wa, Wb, W1, ba, bb, b1)

    npad = ((n + 8 * _NS - 1) // (8 * _NS)) * (8 * _NS)
    quant = _NW * _SG * _CH
    epad = ((e + quant - 1) // quant) * quant
    src = edge_index[0].astype(jnp.int32)
    dst = edge_index[1].astype(jnp.int32)
    if epad != e:
        fill = jnp.arange(epad - e, dtype=jnp.int32)
        src = jnp.concatenate([src, fill % n])
        dst = jnp.concatenate([dst, n + fill % (npad - n)])
    nstripe = epad // (_NW * _SG * _CH)
    src4d = src.reshape(_NW, nstripe, _SG, _CH)
    dst4d = dst.reshape(_NW, nstripe, _SG, _CH)
    zeros_nd = jnp.zeros((npad, 128), _F32)

    ESMpart = _esm(ESM_nodea_emb, ESM_nodeb_emb, WFa, WFb, bfold)

    agg2 = _edge_agg(xm, src4d, dst4d, zeros_nd)

    Pa, Pb = _mid(agg2, W_upd, b_upd, W1)

    gch = 128
    ia3d = edge_a.astype(jnp.int32).reshape(_NW, bsz // (_NW * gch), gch)
    ib3d = edge_b.astype(jnp.int32).reshape(_NW, bsz // (_NW * gch), gch)
    GA, GB = _gather_pred(Pa, Pb, ia3d, ib3d)

    return _head(GA, GB, ESMpart, W2, b2)


# speedup vs baseline: 1.0163x; 1.0163x over previous
import functools

import jax
import jax.numpy as jnp
from jax import lax
from jax.experimental import pallas as pl
from jax.experimental.pallas import tpu as pltpu
from jax.experimental.pallas import tpu_sc as plsc

_F32 = jnp.float32
_NC = 2
_NS = 16
_NW = _NC * _NS
_CH = 128


def _prep_body(x_ref, wmsg_ref, wa_ref, wb_ref, w1_ref, ba_ref, bb_ref,
               b1_ref, xm_ref, wfa_ref, wfb_ref, bf_ref):
    xm_ref[...] = jnp.dot(x_ref[...], wmsg_ref[...],
                          preferred_element_type=_F32)
    w1a = w1_ref[256:512, :]
    w1b = w1_ref[512:768, :]
    wfa_ref[...] = jnp.dot(wa_ref[...], w1a, preferred_element_type=_F32)
    wfb_ref[...] = jnp.dot(wb_ref[...], w1b, preferred_element_type=_F32)
    bf_ref[...] = (b1_ref[...]
                   + jnp.dot(ba_ref[...], w1a, preferred_element_type=_F32)
                   + jnp.dot(bb_ref[...], w1b, preferred_element_type=_F32))


def _prep(x, W_msg, Wa, Wb, W1, ba, bb, b1):
    n, d = x.shape
    k = Wa.shape[0]
    return pl.pallas_call(
        _prep_body,
        out_shape=(
            jax.ShapeDtypeStruct((n, d), _F32),
            jax.ShapeDtypeStruct((k, 128), _F32),
            jax.ShapeDtypeStruct((k, 128), _F32),
            jax.ShapeDtypeStruct((1, 128), _F32),
        ),
    )(x, W_msg, Wa, Wb, W1, ba.reshape(1, -1), bb.reshape(1, -1),
      b1.reshape(1, -1))


def _mid_body(agg_ref, wupd_ref, bupd_ref, w1_ref, pa_ref, pb_ref):
    s = agg_ref[0] + agg_ref[1]
    r = jnp.maximum(
        jnp.dot(s, wupd_ref[...], preferred_element_type=_F32)
        + bupd_ref[...], 0.0)
    pa_ref[...] = jnp.dot(r, w1_ref[0:128, :], preferred_element_type=_F32)
    pb_ref[...] = jnp.dot(r, w1_ref[128:256, :], preferred_element_type=_F32)


def _mid(agg2, W_upd, b_upd, W1):
    n = agg2.shape[1]
    return pl.pallas_call(
        _mid_body,
        out_shape=(
            jax.ShapeDtypeStruct((n, 128), _F32),
            jax.ShapeDtypeStruct((n, 128), _F32),
        ),
    )(agg2, W_upd, b_upd.reshape(1, -1), W1)


def _esm_body(ea_ref, eb_ref, wfa_ref, wfb_ref, bf_ref, out_ref):
    out_ref[...] = (
        jnp.dot(ea_ref[...], wfa_ref[...], preferred_element_type=_F32)
        + jnp.dot(eb_ref[...], wfb_ref[...], preferred_element_type=_F32)
        + bf_ref[...])


def _esm(ESMa, ESMb, WFa, WFb, bfold):
    b, k = ESMa.shape
    bm = 1024
    grid = (b // bm,)
    return pl.pallas_call(
        _esm_body,
        grid=grid,
        in_specs=[
            pl.BlockSpec((bm, k), lambda i: (i, 0)),
            pl.BlockSpec((bm, k), lambda i: (i, 0)),
            pl.BlockSpec((k, 128), lambda i: (0, 0)),
            pl.BlockSpec((k, 128), lambda i: (0, 0)),
            pl.BlockSpec((1, 128), lambda i: (0, 0)),
        ],
        out_specs=pl.BlockSpec((bm, 128), lambda i: (i, 0)),
        out_shape=jax.ShapeDtypeStruct((b, 128), _F32),
    )(ESMa, ESMb, WFa, WFb, bfold)


def _head_body(ga_ref, gb_ref, ep_ref, w2_ref, b2_ref, out_ref):
    h = jnp.maximum(ga_ref[...] + gb_ref[...] + ep_ref[...], 0.0)
    out_ref[...] = (jnp.dot(h, w2_ref[...], preferred_element_type=_F32)
                    + b2_ref[...])


def _head(GA, GB, ESMpart, W2, b2):
    b = GA.shape[0]
    ncls = W2.shape[1]
    bm = 2048
    grid = (b // bm,)
    return pl.pallas_call(
        _head_body,
        grid=grid,
        in_specs=[
            pl.BlockSpec((bm, 128), lambda i: (i, 0)),
            pl.BlockSpec((bm, 128), lambda i: (i, 0)),
            pl.BlockSpec((bm, 128), lambda i: (i, 0)),
            pl.BlockSpec((128, ncls), lambda i: (0, 0)),
            pl.BlockSpec((1, ncls), lambda i: (0, 0)),
        ],
        out_specs=pl.BlockSpec((bm, ncls), lambda i: (i, 0)),
        out_shape=jax.ShapeDtypeStruct((b, ncls), _F32),
    )(GA, GB, ESMpart, W2, b2.reshape(1, -1))


_SG = 16


def _edge_agg(xm, src4d, dst4d, zeros_nd):
    npad = zeros_nd.shape[0]
    nstripe = src4d.shape[1]
    rpt = npad // _NS
    mesh = plsc.VectorSubcoreMesh(core_axis_name="c", subcore_axis_name="s",
                                  num_cores=_NC, num_subcores=_NS)

    @functools.partial(
        pl.kernel,
        out_type=jax.ShapeDtypeStruct((_NC, npad, 128), _F32),
        mesh=mesh,
        scratch_types=[
            pltpu.VMEM((2, _SG, _CH), jnp.int32),
            pltpu.VMEM((2, _SG, _CH), jnp.int32),
            pltpu.VMEM((_CH, 128), _F32),
            pltpu.VMEM((_CH, 128), _F32),
            pltpu.SemaphoreType.DMA,
            pltpu.SemaphoreType.DMA,
            pltpu.SemaphoreType.DMA,
            pltpu.VMEM_SHARED((npad, 128), _F32),
        ],
    )
    def k(xm_hbm, src_hbm, dst_hbm, zero_hbm, agg_hbm,
          srcv, dstv, rows0, rows1, gsem0, gsem1, isem, acc):
        c = lax.axis_index("c")
        s = lax.axis_index("s")
        w = c * _NS + s
        pltpu.sync_copy(zero_hbm.at[pl.ds(s * rpt, rpt)],
                        acc.at[pl.ds(s * rpt, rpt)])
        plsc.subcore_barrier()
        pltpu.sync_copy(src_hbm.at[w].at[0], srcv.at[0])
        pltpu.sync_copy(dst_hbm.at[w].at[0], dstv.at[0])

        def stripe(t, carry):
            slot = lax.rem(t, 2)
            nxt = lax.rem(t + 1, 2)
            sv = srcv.at[slot]
            dv = dstv.at[slot]

            @pl.when(t + 1 < nstripe)
            def _():
                pltpu.async_copy(src_hbm.at[w].at[t + 1], srcv.at[nxt], isem)
                pltpu.async_copy(dst_hbm.at[w].at[t + 1], dstv.at[nxt], isem)

            pltpu.async_copy(xm_hbm.at[sv.at[0]], rows0, gsem0)
            pltpu.async_copy(xm_hbm.at[sv.at[1]], rows1, gsem1)

            def pair(p, carry2):
                j0 = 2 * p
                pltpu.make_async_copy(xm_hbm.at[sv.at[j0]], rows0,
                                      gsem0).wait()
                pltpu.sync_copy(rows0, acc.at[dv.at[j0]], add=True)

                @pl.when(j0 + 2 < _SG)
                def _():
                    pltpu.async_copy(xm_hbm.at[sv.at[j0 + 2]], rows0, gsem0)

                pltpu.make_async_copy(xm_hbm.at[sv.at[j0 + 1]], rows1,
                                      gsem1).wait()
                pltpu.sync_copy(rows1, acc.at[dv.at[j0 + 1]], add=True)

                @pl.when(j0 + 3 < _SG)
                def _():
                    pltpu.async_copy(xm_hbm.at[sv.at[j0 + 3]], rows1, gsem1)

                return carry2

            lax.fori_loop(0, _SG // 2, pair, 0)

            @pl.when(t + 1 < nstripe)
            def _():
                pltpu.make_async_copy(src_hbm.at[w].at[t + 1], srcv.at[nxt],
                                      isem).wait()
                pltpu.make_async_copy(dst_hbm.at[w].at[t + 1], dstv.at[nxt],
                                      isem).wait()

            return carry

        lax.fori_loop(0, nstripe, stripe, 0)
        plsc.subcore_barrier()
        pltpu.sync_copy(acc.at[pl.ds(s * rpt, rpt)],
                        agg_hbm.at[c].at[pl.ds(s * rpt, rpt)])

    return k(xm, src4d, dst4d, zeros_nd)


def _gather_pred(Pa, Pb, ia2d, ib2d):
    b = ia2d.shape[0] * ia2d.shape[1] * ia2d.shape[2]
    nchunk_w = ia2d.shape[1]
    gch = ia2d.shape[2]
    mesh = plsc.VectorSubcoreMesh(core_axis_name="c", subcore_axis_name="s",
                                  num_cores=_NC, num_subcores=_NS)

    @functools.partial(
        pl.kernel,
        out_type=(jax.ShapeDtypeStruct((b, 128), _F32),
                  jax.ShapeDtypeStruct((b, 128), _F32)),
        mesh=mesh,
        scratch_types=[
            pltpu.VMEM((nchunk_w, gch), jnp.int32),
            pltpu.VMEM((nchunk_w, gch), jnp.int32),
            pltpu.VMEM((gch, 128), _F32),
            pltpu.VMEM((gch, 128), _F32),
            pltpu.SemaphoreType.DMA,
            pltpu.SemaphoreType.DMA,
        ],
    )
    def k(pa_hbm, pb_hbm, ia_hbm, ib_hbm, ga_hbm, gb_hbm,
          iav, ibv, bufa, bufb, sema, semb):
        c = lax.axis_index("c")
        s = lax.axis_index("s")
        w = c * _NS + s
        base = w * nchunk_w
        pltpu.sync_copy(ia_hbm.at[w], iav)
        pltpu.sync_copy(ib_hbm.at[w], ibv)

        def body(j, carry):
            ca = pltpu.async_copy(pa_hbm.at[iav.at[j]], bufa, sema)
            cb = pltpu.async_copy(pb_hbm.at[ibv.at[j]], bufb, semb)
            ca.wait()
            cb.wait()
            row0 = (base + j) * gch
            pltpu.sync_copy(bufa, ga_hbm.at[pl.ds(row0, gch)])
            pltpu.sync_copy(bufb, gb_hbm.at[pl.ds(row0, gch)])
            return carry

        lax.fori_loop(0, nchunk_w, body, 0)

    return k(Pa, Pb, ia2d, ib2d)


def kernel(x, edge_index, edge_a, edge_b, ESM_nodea_emb, ESM_nodeb_emb,
           W_msg, W_upd, b_upd, Wa, ba, Wb, bb, W1, b1, W2, b2):
    n = x.shape[0]
    e = edge_index.shape[1]
    bsz = edge_a.shape[0]

    xm, WFa, WFb, bfold = _prep(x, W_msg, Wa, Wb, W1, ba, bb, b1)

    npad = ((n + 8 * _NS - 1) // (8 * _NS)) * (8 * _NS)
    quant = _NW * _SG * _CH
    epad = ((e + quant - 1) // quant) * quant
    src = edge_index[0].astype(jnp.int32)
    dst = edge_index[1].astype(jnp.int32)
    if epad != e:
        fill = jnp.arange(epad - e, dtype=jnp.int32)
        src = jnp.concatenate([src, fill % n])
        dst = jnp.concatenate([dst, n + fill % (npad - n)])
    nstripe = epad // (_NW * _SG * _CH)
    src4d = src.reshape(_NW, nstripe, _SG, _CH)
    dst4d = dst.reshape(_NW, nstripe, _SG, _CH)
    zeros_nd = jnp.zeros((npad, 128), _F32)

    agg2 = _edge_agg(xm, src4d, dst4d, zeros_nd)

    ESMpart = _esm(ESM_nodea_emb, ESM_nodeb_emb, WFa, WFb, bfold)

    Pa, Pb = _mid(agg2, W_upd, b_upd, W1)

    gch = 128
    ia3d = edge_a.astype(jnp.int32).reshape(_NW, bsz // (_NW * gch), gch)
    ib3d = edge_b.astype(jnp.int32).reshape(_NW, bsz // (_NW * gch), gch)
    GA, GB = _gather_pred(Pa, Pb, ia3d, ib3d)

    return _head(GA, GB, ESMpart, W2, b2)


# speedup vs baseline: 1.0276x; 1.0111x over previous
import functools

import jax
import jax.numpy as jnp
from jax import lax
from jax.experimental import pallas as pl
from jax.experimental.pallas import tpu as pltpu
from jax.experimental.pallas import tpu_sc as plsc

_F32 = jnp.float32
_NC = 2
_NS = 16
_NW = _NC * _NS
_CH = 128


def _prep_body(x_ref, wmsg_ref, wa_ref, wb_ref, w1_ref, ba_ref, bb_ref,
               b1_ref, xm_ref, wfa_ref, wfb_ref, bf_ref):
    xm_ref[...] = jnp.dot(x_ref[...], wmsg_ref[...],
                          preferred_element_type=_F32)
    w1a = w1_ref[256:512, :]
    w1b = w1_ref[512:768, :]
    wfa_ref[...] = jnp.dot(wa_ref[...], w1a, preferred_element_type=_F32)
    wfb_ref[...] = jnp.dot(wb_ref[...], w1b, preferred_element_type=_F32)
    bf_ref[...] = (b1_ref[...]
                   + jnp.dot(ba_ref[...], w1a, preferred_element_type=_F32)
                   + jnp.dot(bb_ref[...], w1b, preferred_element_type=_F32))


def _prep(x, W_msg, Wa, Wb, W1, ba, bb, b1):
    n, d = x.shape
    k = Wa.shape[0]
    return pl.pallas_call(
        _prep_body,
        out_shape=(
            jax.ShapeDtypeStruct((n, d), _F32),
            jax.ShapeDtypeStruct((k, 128), _F32),
            jax.ShapeDtypeStruct((k, 128), _F32),
            jax.ShapeDtypeStruct((1, 128), _F32),
        ),
    )(x, W_msg, Wa, Wb, W1, ba.reshape(1, -1), bb.reshape(1, -1),
      b1.reshape(1, -1))


def _mid_body(agg_ref, wupd_ref, bupd_ref, w1_ref, pa_ref, pb_ref):
    s = agg_ref[0] + agg_ref[1]
    r = jnp.maximum(
        jnp.dot(s, wupd_ref[...], preferred_element_type=_F32)
        + bupd_ref[...], 0.0)
    pa_ref[...] = jnp.dot(r, w1_ref[0:128, :], preferred_element_type=_F32)
    pb_ref[...] = jnp.dot(r, w1_ref[128:256, :], preferred_element_type=_F32)


def _mid(agg2, W_upd, b_upd, W1):
    n = agg2.shape[1]
    return pl.pallas_call(
        _mid_body,
        out_shape=(
            jax.ShapeDtypeStruct((n, 128), _F32),
            jax.ShapeDtypeStruct((n, 128), _F32),
        ),
    )(agg2, W_upd, b_upd.reshape(1, -1), W1)


def _head_body(ga_ref, gb_ref, ea_ref, eb_ref, wfa_ref, wfb_ref, bf_ref,
               w2_ref, b2_ref, out_ref):
    ep = (jnp.dot(ea_ref[...], wfa_ref[...], preferred_element_type=_F32)
          + jnp.dot(eb_ref[...], wfb_ref[...], preferred_element_type=_F32)
          + bf_ref[...])
    h = jnp.maximum(ga_ref[...] + gb_ref[...] + ep, 0.0)
    out_ref[...] = (jnp.dot(h, w2_ref[...], preferred_element_type=_F32)
                    + b2_ref[...])


def _head(GA, GB, ESMa, ESMb, WFa, WFb, bfold, W2, b2):
    b, k = ESMa.shape
    ncls = W2.shape[1]
    bm = 1024
    grid = (b // bm,)
    return pl.pallas_call(
        _head_body,
        grid=grid,
        in_specs=[
            pl.BlockSpec((bm, 128), lambda i: (i, 0)),
            pl.BlockSpec((bm, 128), lambda i: (i, 0)),
            pl.BlockSpec((bm, k), lambda i: (i, 0)),
            pl.BlockSpec((bm, k), lambda i: (i, 0)),
            pl.BlockSpec((k, 128), lambda i: (0, 0)),
            pl.BlockSpec((k, 128), lambda i: (0, 0)),
            pl.BlockSpec((1, 128), lambda i: (0, 0)),
            pl.BlockSpec((128, ncls), lambda i: (0, 0)),
            pl.BlockSpec((1, ncls), lambda i: (0, 0)),
        ],
        out_specs=pl.BlockSpec((bm, ncls), lambda i: (i, 0)),
        out_shape=jax.ShapeDtypeStruct((b, ncls), _F32),
    )(GA, GB, ESMa, ESMb, WFa, WFb, bfold, W2, b2.reshape(1, -1))


_SG = 16


def _edge_agg(xm, src4d, dst4d, zeros_nd):
    npad = zeros_nd.shape[0]
    nstripe = src4d.shape[1]
    rpt = npad // _NS
    mesh = plsc.VectorSubcoreMesh(core_axis_name="c", subcore_axis_name="s",
                                  num_cores=_NC, num_subcores=_NS)

    @functools.partial(
        pl.kernel,
        out_type=jax.ShapeDtypeStruct((_NC, npad, 128), _F32),
        mesh=mesh,
        scratch_types=[
            pltpu.VMEM((2, _SG, _CH), jnp.int32),
            pltpu.VMEM((2, _SG, _CH), jnp.int32),
            pltpu.VMEM((_CH, 128), _F32),
            pltpu.VMEM((_CH, 128), _F32),
            pltpu.SemaphoreType.DMA,
            pltpu.SemaphoreType.DMA,
            pltpu.SemaphoreType.DMA,
            pltpu.VMEM_SHARED((npad, 128), _F32),
        ],
    )
    def k(xm_hbm, src_hbm, dst_hbm, zero_hbm, agg_hbm,
          srcv, dstv, rows0, rows1, gsem0, gsem1, isem, acc):
        c = lax.axis_index("c")
        s = lax.axis_index("s")
        w = c * _NS + s
        pltpu.sync_copy(zero_hbm.at[pl.ds(s * rpt, rpt)],
                        acc.at[pl.ds(s * rpt, rpt)])
        plsc.subcore_barrier()
        pltpu.sync_copy(src_hbm.at[w].at[0], srcv.at[0])
        pltpu.sync_copy(dst_hbm.at[w].at[0], dstv.at[0])

        def stripe(t, carry):
            slot = lax.rem(t, 2)
            nxt = lax.rem(t + 1, 2)
            sv = srcv.at[slot]
            dv = dstv.at[slot]

            @pl.when(t + 1 < nstripe)
            def _():
                pltpu.async_copy(src_hbm.at[w].at[t + 1], srcv.at[nxt], isem)
                pltpu.async_copy(dst_hbm.at[w].at[t + 1], dstv.at[nxt], isem)

            pltpu.async_copy(xm_hbm.at[sv.at[0]], rows0, gsem0)
            pltpu.async_copy(xm_hbm.at[sv.at[1]], rows1, gsem1)

            def pair(p, carry2):
                j0 = 2 * p
                pltpu.make_async_copy(xm_hbm.at[sv.at[j0]], rows0,
                                      gsem0).wait()
                pltpu.sync_copy(rows0, acc.at[dv.at[j0]], add=True)

                @pl.when(j0 + 2 < _SG)
                def _():
                    pltpu.async_copy(xm_hbm.at[sv.at[j0 + 2]], rows0, gsem0)

                pltpu.make_async_copy(xm_hbm.at[sv.at[j0 + 1]], rows1,
                                      gsem1).wait()
                pltpu.sync_copy(rows1, acc.at[dv.at[j0 + 1]], add=True)

                @pl.when(j0 + 3 < _SG)
                def _():
                    pltpu.async_copy(xm_hbm.at[sv.at[j0 + 3]], rows1, gsem1)

                return carry2

            lax.fori_loop(0, _SG // 2, pair, 0)

            @pl.when(t + 1 < nstripe)
            def _():
                pltpu.make_async_copy(src_hbm.at[w].at[t + 1], srcv.at[nxt],
                                      isem).wait()
                pltpu.make_async_copy(dst_hbm.at[w].at[t + 1], dstv.at[nxt],
                                      isem).wait()

            return carry

        lax.fori_loop(0, nstripe, stripe, 0)
        plsc.subcore_barrier()
        pltpu.sync_copy(acc.at[pl.ds(s * rpt, rpt)],
                        agg_hbm.at[c].at[pl.ds(s * rpt, rpt)])

    return k(xm, src4d, dst4d, zeros_nd)


def _gather_pred(Pa, Pb, ia2d, ib2d):
    b = ia2d.shape[0] * ia2d.shape[1] * ia2d.shape[2]
    nchunk_w = ia2d.shape[1]
    gch = ia2d.shape[2]
    mesh = plsc.VectorSubcoreMesh(core_axis_name="c", subcore_axis_name="s",
                                  num_cores=_NC, num_subcores=_NS)

    @functools.partial(
        pl.kernel,
        out_type=(jax.ShapeDtypeStruct((b, 128), _F32),
                  jax.ShapeDtypeStruct((b, 128), _F32)),
        mesh=mesh,
        scratch_types=[
            pltpu.VMEM((nchunk_w, gch), jnp.int32),
            pltpu.VMEM((nchunk_w, gch), jnp.int32),
            pltpu.VMEM((gch, 128), _F32),
            pltpu.VMEM((gch, 128), _F32),
            pltpu.SemaphoreType.DMA,
            pltpu.SemaphoreType.DMA,
        ],
    )
    def k(pa_hbm, pb_hbm, ia_hbm, ib_hbm, ga_hbm, gb_hbm,
          iav, ibv, bufa, bufb, sema, semb):
        c = lax.axis_index("c")
        s = lax.axis_index("s")
        w = c * _NS + s
        base = w * nchunk_w
        pltpu.sync_copy(ia_hbm.at[w], iav)
        pltpu.sync_copy(ib_hbm.at[w], ibv)

        def body(j, carry):
            ca = pltpu.async_copy(pa_hbm.at[iav.at[j]], bufa, sema)
            cb = pltpu.async_copy(pb_hbm.at[ibv.at[j]], bufb, semb)
            ca.wait()
            cb.wait()
            row0 = (base + j) * gch
            pltpu.sync_copy(bufa, ga_hbm.at[pl.ds(row0, gch)])
            pltpu.sync_copy(bufb, gb_hbm.at[pl.ds(row0, gch)])
            return carry

        lax.fori_loop(0, nchunk_w, body, 0)

    return k(Pa, Pb, ia2d, ib2d)


def kernel(x, edge_index, edge_a, edge_b, ESM_nodea_emb, ESM_nodeb_emb,
           W_msg, W_upd, b_upd, Wa, ba, Wb, bb, W1, b1, W2, b2):
    n = x.shape[0]
    e = edge_index.shape[1]
    bsz = edge_a.shape[0]

    xm, WFa, WFb, bfold = _prep(x, W_msg, Wa, Wb, W1, ba, bb, b1)

    npad = ((n + 8 * _NS - 1) // (8 * _NS)) * (8 * _NS)
    quant = _NW * _SG * _CH
    epad = ((e + quant - 1) // quant) * quant
    src = edge_index[0].astype(jnp.int32)
    dst = edge_index[1].astype(jnp.int32)
    if epad != e:
        fill = jnp.arange(epad - e, dtype=jnp.int32)
        src = jnp.concatenate([src, fill % n])
        dst = jnp.concatenate([dst, n + fill % (npad - n)])
    nstripe = epad // (_NW * _SG * _CH)
    src4d = src.reshape(_NW, nstripe, _SG, _CH)
    dst4d = dst.reshape(_NW, nstripe, _SG, _CH)
    zeros_nd = jnp.zeros((npad, 128), _F32)

    agg2 = _edge_agg(xm, src4d, dst4d, zeros_nd)

    Pa, Pb = _mid(agg2, W_upd, b_upd, W1)

    gch = 128
    ia3d = edge_a.astype(jnp.int32).reshape(_NW, bsz // (_NW * gch), gch)
    ib3d = edge_b.astype(jnp.int32).reshape(_NW, bsz // (_NW * gch), gch)
    GA, GB = _gather_pred(Pa, Pb, ia3d, ib3d)

    return _head(GA, GB, ESM_nodea_emb, ESM_nodeb_emb, WFa, WFb,
                 bfold, W2, b2)


# speedup vs baseline: 1.0276x; 1.0000x over previous
import functools

import jax
import jax.numpy as jnp
from jax import lax
from jax.experimental import pallas as pl
from jax.experimental.pallas import tpu as pltpu
from jax.experimental.pallas import tpu_sc as plsc

_F32 = jnp.float32
_NC = 2
_NS = 16
_NW = _NC * _NS
_CH = 128


def _prep_body(x_ref, wmsg_ref, wa_ref, wb_ref, w1_ref, ba_ref, bb_ref,
               b1_ref, xm_ref, wfa_ref, wfb_ref, bf_ref):
    xm_ref[...] = jnp.dot(x_ref[...], wmsg_ref[...],
                          preferred_element_type=_F32)
    w1a = w1_ref[256:512, :]
    w1b = w1_ref[512:768, :]
    wfa_ref[...] = jnp.dot(wa_ref[...], w1a, preferred_element_type=_F32)
    wfb_ref[...] = jnp.dot(wb_ref[...], w1b, preferred_element_type=_F32)
    bf_ref[...] = (b1_ref[...]
                   + jnp.dot(ba_ref[...], w1a, preferred_element_type=_F32)
                   + jnp.dot(bb_ref[...], w1b, preferred_element_type=_F32))


def _prep(x, W_msg, Wa, Wb, W1, ba, bb, b1):
    n, d = x.shape
    k = Wa.shape[0]
    return pl.pallas_call(
        _prep_body,
        out_shape=(
            jax.ShapeDtypeStruct((n, d), _F32),
            jax.ShapeDtypeStruct((k, 128), _F32),
            jax.ShapeDtypeStruct((k, 128), _F32),
            jax.ShapeDtypeStruct((1, 128), _F32),
        ),
    )(x, W_msg, Wa, Wb, W1, ba.reshape(1, -1), bb.reshape(1, -1),
      b1.reshape(1, -1))


def _mid_body(agg_ref, wupd_ref, bupd_ref, w1_ref, pa_ref, pb_ref):
    s = agg_ref[0] + agg_ref[1]
    r = jnp.maximum(
        jnp.dot(s, wupd_ref[...], preferred_element_type=_F32)
        + bupd_ref[...], 0.0)
    pa_ref[...] = jnp.dot(r, w1_ref[0:128, :], preferred_element_type=_F32)
    pb_ref[...] = jnp.dot(r, w1_ref[128:256, :], preferred_element_type=_F32)


def _mid(agg2, W_upd, b_upd, W1):
    n = agg2.shape[1]
    return pl.pallas_call(
        _mid_body,
        out_shape=(
            jax.ShapeDtypeStruct((n, 128), _F32),
            jax.ShapeDtypeStruct((n, 128), _F32),
        ),
    )(agg2, W_upd, b_upd.reshape(1, -1), W1)


def _head_body(ga_ref, gb_ref, ea_ref, eb_ref, wfa_ref, wfb_ref, bf_ref,
               w2_ref, b2_ref, out_ref):
    ep = (jnp.dot(ea_ref[...], wfa_ref[...], preferred_element_type=_F32)
          + jnp.dot(eb_ref[...], wfb_ref[...], preferred_element_type=_F32)
          + bf_ref[...])
    h = jnp.maximum(ga_ref[...] + gb_ref[...] + ep, 0.0)
    out_ref[...] = (jnp.dot(h, w2_ref[...], preferred_element_type=_F32)
                    + b2_ref[...])


def _head(GA, GB, ESMa, ESMb, WFa, WFb, bfold, W2, b2):
    b, k = ESMa.shape
    ncls = W2.shape[1]
    bm = 1024
    grid = (b // bm,)
    return pl.pallas_call(
        _head_body,
        grid=grid,
        in_specs=[
            pl.BlockSpec((bm, 128), lambda i: (i, 0)),
            pl.BlockSpec((bm, 128), lambda i: (i, 0)),
            pl.BlockSpec((bm, k), lambda i: (i, 0)),
            pl.BlockSpec((bm, k), lambda i: (i, 0)),
            pl.BlockSpec((k, 128), lambda i: (0, 0)),
            pl.BlockSpec((k, 128), lambda i: (0, 0)),
            pl.BlockSpec((1, 128), lambda i: (0, 0)),
            pl.BlockSpec((128, ncls), lambda i: (0, 0)),
            pl.BlockSpec((1, ncls), lambda i: (0, 0)),
        ],
        out_specs=pl.BlockSpec((bm, ncls), lambda i: (i, 0)),
        out_shape=jax.ShapeDtypeStruct((b, ncls), _F32),
    )(GA, GB, ESMa, ESMb, WFa, WFb, bfold, W2, b2.reshape(1, -1))


_SG = 20


def _edge_agg(xm, src4d, dst4d, zeros_nd):
    npad = zeros_nd.shape[0]
    nstripe = src4d.shape[1]
    rpt = npad // _NS
    mesh = plsc.VectorSubcoreMesh(core_axis_name="c", subcore_axis_name="s",
                                  num_cores=_NC, num_subcores=_NS)

    @functools.partial(
        pl.kernel,
        out_type=jax.ShapeDtypeStruct((_NC, npad, 128), _F32),
        mesh=mesh,
        scratch_types=[
            pltpu.VMEM((2, _SG, _CH), jnp.int32),
            pltpu.VMEM((2, _SG, _CH), jnp.int32),
            pltpu.VMEM((_CH, 128), _F32),
            pltpu.VMEM((_CH, 128), _F32),
            pltpu.SemaphoreType.DMA,
            pltpu.SemaphoreType.DMA,
            pltpu.SemaphoreType.DMA,
            pltpu.VMEM_SHARED((npad, 128), _F32),
        ],
    )
    def k(xm_hbm, src_hbm, dst_hbm, zero_hbm, agg_hbm,
          srcv, dstv, rows0, rows1, gsem0, gsem1, isem, acc):
        c = lax.axis_index("c")
        s = lax.axis_index("s")
        w = c * _NS + s
        pltpu.sync_copy(zero_hbm.at[pl.ds(s * rpt, rpt)],
                        acc.at[pl.ds(s * rpt, rpt)])
        plsc.subcore_barrier()
        pltpu.sync_copy(src_hbm.at[w].at[0], srcv.at[0])
        pltpu.sync_copy(dst_hbm.at[w].at[0], dstv.at[0])

        def stripe(t, carry):
            slot = lax.rem(t, 2)
            nxt = lax.rem(t + 1, 2)
            sv = srcv.at[slot]
            dv = dstv.at[slot]

            @pl.when(t + 1 < nstripe)
            def _():
                pltpu.async_copy(src_hbm.at[w].at[t + 1], srcv.at[nxt], isem)
                pltpu.async_copy(dst_hbm.at[w].at[t + 1], dstv.at[nxt], isem)

            pltpu.async_copy(xm_hbm.at[sv.at[0]], rows0, gsem0)
            pltpu.async_copy(xm_hbm.at[sv.at[1]], rows1, gsem1)

            def pair(p, carry2):
                j0 = 2 * p
                pltpu.make_async_copy(xm_hbm.at[sv.at[j0]], rows0,
                                      gsem0).wait()
                pltpu.sync_copy(rows0, acc.at[dv.at[j0]], add=True)

                @pl.when(j0 + 2 < _SG)
                def _():
                    pltpu.async_copy(xm_hbm.at[sv.at[j0 + 2]], rows0, gsem0)

                pltpu.make_async_copy(xm_hbm.at[sv.at[j0 + 1]], rows1,
                                      gsem1).wait()
                pltpu.sync_copy(rows1, acc.at[dv.at[j0 + 1]], add=True)

                @pl.when(j0 + 3 < _SG)
                def _():
                    pltpu.async_copy(xm_hbm.at[sv.at[j0 + 3]], rows1, gsem1)

                return carry2

            lax.fori_loop(0, _SG // 2, pair, 0)

            @pl.when(t + 1 < nstripe)
            def _():
                pltpu.make_async_copy(src_hbm.at[w].at[t + 1], srcv.at[nxt],
                                      isem).wait()
                pltpu.make_async_copy(dst_hbm.at[w].at[t + 1], dstv.at[nxt],
                                      isem).wait()

            return carry

        lax.fori_loop(0, nstripe, stripe, 0)
        plsc.subcore_barrier()
        pltpu.sync_copy(acc.at[pl.ds(s * rpt, rpt)],
                        agg_hbm.at[c].at[pl.ds(s * rpt, rpt)])

    return k(xm, src4d, dst4d, zeros_nd)


def _gather_pred(Pa, Pb, ia2d, ib2d):
    b = ia2d.shape[0] * ia2d.shape[1] * ia2d.shape[2]
    nchunk_w = ia2d.shape[1]
    gch = ia2d.shape[2]
    mesh = plsc.VectorSubcoreMesh(core_axis_name="c", subcore_axis_name="s",
                                  num_cores=_NC, num_subcores=_NS)

    @functools.partial(
        pl.kernel,
        out_type=(jax.ShapeDtypeStruct((b, 128), _F32),
                  jax.ShapeDtypeStruct((b, 128), _F32)),
        mesh=mesh,
        scratch_types=[
            pltpu.VMEM((nchunk_w, gch), jnp.int32),
            pltpu.VMEM((nchunk_w, gch), jnp.int32),
            pltpu.VMEM((gch, 128), _F32),
            pltpu.VMEM((gch, 128), _F32),
            pltpu.SemaphoreType.DMA,
            pltpu.SemaphoreType.DMA,
        ],
    )
    def k(pa_hbm, pb_hbm, ia_hbm, ib_hbm, ga_hbm, gb_hbm,
          iav, ibv, bufa, bufb, sema, semb):
        c = lax.axis_index("c")
        s = lax.axis_index("s")
        w = c * _NS + s
        base = w * nchunk_w
        pltpu.sync_copy(ia_hbm.at[w], iav)
        pltpu.sync_copy(ib_hbm.at[w], ibv)

        def body(j, carry):
            ca = pltpu.async_copy(pa_hbm.at[iav.at[j]], bufa, sema)
            cb = pltpu.async_copy(pb_hbm.at[ibv.at[j]], bufb, semb)
            ca.wait()
            cb.wait()
            row0 = (base + j) * gch
            pltpu.sync_copy(bufa, ga_hbm.at[pl.ds(row0, gch)])
            pltpu.sync_copy(bufb, gb_hbm.at[pl.ds(row0, gch)])
            return carry

        lax.fori_loop(0, nchunk_w, body, 0)

    return k(Pa, Pb, ia2d, ib2d)


def kernel(x, edge_index, edge_a, edge_b, ESM_nodea_emb, ESM_nodeb_emb,
           W_msg, W_upd, b_upd, Wa, ba, Wb, bb, W1, b1, W2, b2):
    n = x.shape[0]
    e = edge_index.shape[1]
    bsz = edge_a.shape[0]

    xm, WFa, WFb, bfold = _prep(x, W_msg, Wa, Wb, W1, ba, bb, b1)

    npad = ((n + 8 * _NS - 1) // (8 * _NS)) * (8 * _NS)
    quant = _NW * _SG * _CH
    epad = ((e + quant - 1) // quant) * quant
    src = edge_index[0].astype(jnp.int32)
    dst = edge_index[1].astype(jnp.int32)
    if epad != e:
        fill = jnp.arange(epad - e, dtype=jnp.int32)
        src = jnp.concatenate([src, fill % n])
        dst = jnp.concatenate([dst, n + fill % (npad - n)])
    nstripe = epad // (_NW * _SG * _CH)
    src4d = src.reshape(_NW, nstripe, _SG, _CH)
    dst4d = dst.reshape(_NW, nstripe, _SG, _CH)
    zeros_nd = jnp.zeros((npad, 128), _F32)

    agg2 = _edge_agg(xm, src4d, dst4d, zeros_nd)

    Pa, Pb = _mid(agg2, W_upd, b_upd, W1)

    gch = 128
    ia3d = edge_a.astype(jnp.int32).reshape(_NW, bsz // (_NW * gch), gch)
    ib3d = edge_b.astype(jnp.int32).reshape(_NW, bsz // (_NW * gch), gch)
    GA, GB = _gather_pred(Pa, Pb, ia3d, ib3d)

    return _head(GA, GB, ESM_nodea_emb, ESM_nodeb_emb, WFa, WFb,
                 bfold, W2, b2)


# speedup vs baseline: 1.0307x; 1.0030x over previous
import functools

import jax
import jax.numpy as jnp
from jax import lax
from jax.experimental import pallas as pl
from jax.experimental.pallas import tpu as pltpu
from jax.experimental.pallas import tpu_sc as plsc

_F32 = jnp.float32
_NC = 2
_NS = 16
_NW = _NC * _NS
_CH = 128


def _prep_body(x_ref, wmsg_ref, wa_ref, wb_ref, w1_ref, ba_ref, bb_ref,
               b1_ref, xm_ref, wfa_ref, wfb_ref, bf_ref):
    xm_ref[...] = jnp.dot(x_ref[...], wmsg_ref[...],
                          preferred_element_type=_F32)
    w1a = w1_ref[256:512, :]
    w1b = w1_ref[512:768, :]
    wfa_ref[...] = jnp.dot(wa_ref[...], w1a, preferred_element_type=_F32)
    wfb_ref[...] = jnp.dot(wb_ref[...], w1b, preferred_element_type=_F32)
    bf_ref[...] = (b1_ref[...]
                   + jnp.dot(ba_ref[...], w1a, preferred_element_type=_F32)
                   + jnp.dot(bb_ref[...], w1b, preferred_element_type=_F32))


def _prep(x, W_msg, Wa, Wb, W1, ba, bb, b1):
    n, d = x.shape
    k = Wa.shape[0]
    return pl.pallas_call(
        _prep_body,
        out_shape=(
            jax.ShapeDtypeStruct((n, d), _F32),
            jax.ShapeDtypeStruct((k, 128), _F32),
            jax.ShapeDtypeStruct((k, 128), _F32),
            jax.ShapeDtypeStruct((1, 128), _F32),
        ),
    )(x, W_msg, Wa, Wb, W1, ba.reshape(1, -1), bb.reshape(1, -1),
      b1.reshape(1, -1))


def _mid_body(agg_ref, wupd_ref, bupd_ref, w1_ref, pa_ref, pb_ref):
    s = agg_ref[0] + agg_ref[1]
    r = jnp.maximum(
        jnp.dot(s, wupd_ref[...], preferred_element_type=_F32)
        + bupd_ref[...], 0.0)
    pa_ref[...] = jnp.dot(r, w1_ref[0:128, :], preferred_element_type=_F32)
    pb_ref[...] = jnp.dot(r, w1_ref[128:256, :], preferred_element_type=_F32)


def _mid(agg2, W_upd, b_upd, W1):
    n = agg2.shape[1]
    return pl.pallas_call(
        _mid_body,
        out_shape=(
            jax.ShapeDtypeStruct((n, 128), _F32),
            jax.ShapeDtypeStruct((n, 128), _F32),
        ),
    )(agg2, W_upd, b_upd.reshape(1, -1), W1)


def _head_body(ga_ref, gb_ref, ea_ref, eb_ref, wfa_ref, wfb_ref, bf_ref,
               w2_ref, b2_ref, out_ref):
    ep = (jnp.dot(ea_ref[...], wfa_ref[...], preferred_element_type=_F32)
          + jnp.dot(eb_ref[...], wfb_ref[...], preferred_element_type=_F32)
          + bf_ref[...])
    h = jnp.maximum(ga_ref[...] + gb_ref[...] + ep, 0.0)
    out_ref[...] = (jnp.dot(h, w2_ref[...], preferred_element_type=_F32)
                    + b2_ref[...])


def _head(GA, GB, ESMa, ESMb, WFa, WFb, bfold, W2, b2):
    b, k = ESMa.shape
    ncls = W2.shape[1]
    bm = 1024
    grid = (b // bm,)
    return pl.pallas_call(
        _head_body,
        grid=grid,
        in_specs=[
            pl.BlockSpec((bm, 128), lambda i: (i, 0)),
            pl.BlockSpec((bm, 128), lambda i: (i, 0)),
            pl.BlockSpec((bm, k), lambda i: (i, 0)),
            pl.BlockSpec((bm, k), lambda i: (i, 0)),
            pl.BlockSpec((k, 128), lambda i: (0, 0)),
            pl.BlockSpec((k, 128), lambda i: (0, 0)),
            pl.BlockSpec((1, 128), lambda i: (0, 0)),
            pl.BlockSpec((128, ncls), lambda i: (0, 0)),
            pl.BlockSpec((1, ncls), lambda i: (0, 0)),
        ],
        out_specs=pl.BlockSpec((bm, ncls), lambda i: (i, 0)),
        out_shape=jax.ShapeDtypeStruct((b, ncls), _F32),
    )(GA, GB, ESMa, ESMb, WFa, WFb, bfold, W2, b2.reshape(1, -1))


_SG = 20


def _edge_agg(xm, src4d, dst4d, zeros_nd):
    npad = zeros_nd.shape[0]
    nstripe = src4d.shape[1]
    rpt = npad // _NS
    mesh = plsc.VectorSubcoreMesh(core_axis_name="c", subcore_axis_name="s",
                                  num_cores=_NC, num_subcores=_NS)

    @functools.partial(
        pl.kernel,
        out_type=jax.ShapeDtypeStruct((_NC, npad, 128), _F32),
        mesh=mesh,
        scratch_types=[
            pltpu.VMEM((2, _SG, _CH), jnp.int32),
            pltpu.VMEM((2, _SG, _CH), jnp.int32),
            pltpu.VMEM((_CH, 128), _F32),
            pltpu.VMEM((_CH, 128), _F32),
            pltpu.SemaphoreType.DMA,
            pltpu.SemaphoreType.DMA,
            pltpu.SemaphoreType.DMA,
            pltpu.VMEM_SHARED((npad, 128), _F32),
        ],
    )
    def k(xm_hbm, src_hbm, dst_hbm, zero_hbm, agg_hbm,
          srcv, dstv, rows0, rows1, gsem0, gsem1, isem, acc):
        c = lax.axis_index("c")
        s = lax.axis_index("s")
        w = c * _NS + s
        pltpu.sync_copy(zero_hbm.at[pl.ds(s * rpt, rpt)],
                        acc.at[pl.ds(s * rpt, rpt)])
        plsc.subcore_barrier()
        pltpu.sync_copy(src_hbm.at[w].at[0], srcv.at[0])
        pltpu.sync_copy(dst_hbm.at[w].at[0], dstv.at[0])

        def stripe(t, carry):
            slot = lax.rem(t, 2)
            nxt = lax.rem(t + 1, 2)
            sv = srcv.at[slot]
            dv = dstv.at[slot]

            @pl.when(t + 1 < nstripe)
            def _():
                pltpu.async_copy(src_hbm.at[w].at[t + 1], srcv.at[nxt], isem)
                pltpu.async_copy(dst_hbm.at[w].at[t + 1], dstv.at[nxt], isem)

            pltpu.async_copy(xm_hbm.at[sv.at[0]], rows0, gsem0)
            pltpu.async_copy(xm_hbm.at[sv.at[1]], rows1, gsem1)

            def pair(p, carry2):
                j0 = 2 * p
                pltpu.make_async_copy(xm_hbm.at[sv.at[j0]], rows0,
                                      gsem0).wait()
                pltpu.sync_copy(rows0, acc.at[dv.at[j0]], add=True)

                @pl.when(j0 + 2 < _SG)
                def _():
                    pltpu.async_copy(xm_hbm.at[sv.at[j0 + 2]], rows0, gsem0)

                pltpu.make_async_copy(xm_hbm.at[sv.at[j0 + 1]], rows1,
                                      gsem1).wait()
                pltpu.sync_copy(rows1, acc.at[dv.at[j0 + 1]], add=True)

                @pl.when(j0 + 3 < _SG)
                def _():
                    pltpu.async_copy(xm_hbm.at[sv.at[j0 + 3]], rows1, gsem1)

                return carry2

            lax.fori_loop(0, _SG // 2, pair, 0)

            @pl.when(t + 1 < nstripe)
            def _():
                pltpu.make_async_copy(src_hbm.at[w].at[t + 1], srcv.at[nxt],
                                      isem).wait()
                pltpu.make_async_copy(dst_hbm.at[w].at[t + 1], dstv.at[nxt],
                                      isem).wait()

            return carry

        lax.fori_loop(0, nstripe, stripe, 0)
        plsc.subcore_barrier()
        pltpu.sync_copy(acc.at[pl.ds(s * rpt, rpt)],
                        agg_hbm.at[c].at[pl.ds(s * rpt, rpt)])

    return k(xm, src4d, dst4d, zeros_nd)


def _gather_pred(Pa, Pb, ia2d, ib2d):
    b = ia2d.shape[0] * ia2d.shape[1] * ia2d.shape[2]
    nchunk_w = ia2d.shape[1]
    gch = ia2d.shape[2]
    mesh = plsc.VectorSubcoreMesh(core_axis_name="c", subcore_axis_name="s",
                                  num_cores=_NC, num_subcores=_NS)

    @functools.partial(
        pl.kernel,
        out_type=(jax.ShapeDtypeStruct((b, 128), _F32),
                  jax.ShapeDtypeStruct((b, 128), _F32)),
        mesh=mesh,
        scratch_types=[
            pltpu.VMEM((nchunk_w, gch), jnp.int32),
            pltpu.VMEM((nchunk_w, gch), jnp.int32),
            pltpu.VMEM((2, gch, 128), _F32),
            pltpu.VMEM((2, gch, 128), _F32),
            pltpu.SemaphoreType.DMA,
            pltpu.SemaphoreType.DMA,
        ],
    )
    def k(pa_hbm, pb_hbm, ia_hbm, ib_hbm, ga_hbm, gb_hbm,
          iav, ibv, bufa, bufb, sema, semb):
        c = lax.axis_index("c")
        s = lax.axis_index("s")
        w = c * _NS + s
        base = w * nchunk_w
        pltpu.sync_copy(ia_hbm.at[w], iav)
        pltpu.sync_copy(ib_hbm.at[w], ibv)

        pltpu.async_copy(pa_hbm.at[iav.at[0]], bufa.at[0], sema)
        pltpu.async_copy(pb_hbm.at[ibv.at[0]], bufb.at[0], semb)

        def body(j, carry):
            slot = lax.rem(j, 2)
            nxt = lax.rem(j + 1, 2)
            pltpu.make_async_copy(pa_hbm.at[iav.at[j]], bufa.at[slot],
                                  sema).wait()
            pltpu.make_async_copy(pb_hbm.at[ibv.at[j]], bufb.at[slot],
                                  semb).wait()

            @pl.when(j + 1 < nchunk_w)
            def _():
                pltpu.async_copy(pa_hbm.at[iav.at[j + 1]], bufa.at[nxt], sema)
                pltpu.async_copy(pb_hbm.at[ibv.at[j + 1]], bufb.at[nxt], semb)

            row0 = (base + j) * gch
            pltpu.sync_copy(bufa.at[slot], ga_hbm.at[pl.ds(row0, gch)])
            pltpu.sync_copy(bufb.at[slot], gb_hbm.at[pl.ds(row0, gch)])
            return carry

        lax.fori_loop(0, nchunk_w, body, 0)

    return k(Pa, Pb, ia2d, ib2d)


def kernel(x, edge_index, edge_a, edge_b, ESM_nodea_emb, ESM_nodeb_emb,
           W_msg, W_upd, b_upd, Wa, ba, Wb, bb, W1, b1, W2, b2):
    n = x.shape[0]
    e = edge_index.shape[1]
    bsz = edge_a.shape[0]

    xm, WFa, WFb, bfold = _prep(x, W_msg, Wa, Wb, W1, ba, bb, b1)

    npad = ((n + 8 * _NS - 1) // (8 * _NS)) * (8 * _NS)
    quant = _NW * _SG * _CH
    epad = ((e + quant - 1) // quant) * quant
    src = edge_index[0].astype(jnp.int32)
    dst = edge_index[1].astype(jnp.int32)
    if epad != e:
        fill = jnp.arange(epad - e, dtype=jnp.int32)
        src = jnp.concatenate([src, fill % n])
        dst = jnp.concatenate([dst, n + fill % (npad - n)])
    nstripe = epad // (_NW * _SG * _CH)
    src4d = src.reshape(_NW, nstripe, _SG, _CH)
    dst4d = dst.reshape(_NW, nstripe, _SG, _CH)
    zeros_nd = jnp.zeros((npad, 128), _F32)

    agg2 = _edge_agg(xm, src4d, dst4d, zeros_nd)

    Pa, Pb = _mid(agg2, W_upd, b_upd, W1)

    gch = 128
    ia3d = edge_a.astype(jnp.int32).reshape(_NW, bsz // (_NW * gch), gch)
    ib3d = edge_b.astype(jnp.int32).reshape(_NW, bsz // (_NW * gch), gch)
    GA, GB = _gather_pred(Pa, Pb, ia3d, ib3d)

    return _head(GA, GB, ESM_nodea_emb, ESM_nodeb_emb, WFa, WFb,
                 bfold, W2, b2)
